# Initial kernel scaffold; baseline (speedup 1.0000x reference)
import functools

import jax
import jax.numpy as jnp
from jax import lax
from jax.experimental import pallas as pl
from jax.experimental.pallas import tpu as pltpu
from jax.experimental.pallas import tpu_sc as plsc

_N = 10000
_E = 320000
_D = 128
_NC = 2
_NS = 16
_NW = _NC * _NS
_K = 128
_CHUNKS = 80
_EPAD = _NW * _CHUNKS * _K
_ACC_ROWS = 10240
_ZR = _ACC_ROWS // _NS
_XR = _N // _NS
_DG = 16

_mesh = plsc.VectorSubcoreMesh(core_axis_name="c", subcore_axis_name="s")


def _agg1_body(table, srcs, dsts, ones_h, zrows, zdeg,
               part, degp, acc, dacc, srcv, dstv, rows, onesv, sem):
    c = lax.axis_index("c")
    s = lax.axis_index("s")
    wid = c * _NS + s
    pltpu.sync_copy(zrows, acc.at[pl.ds(s * _ZR, _ZR)])
    pltpu.sync_copy(zdeg, dacc.at[pl.ds(s * _ZR, _ZR)])
    pltpu.sync_copy(ones_h, onesv)
    plsc.subcore_barrier()

    def step(j, carry):
        pltpu.sync_copy(srcs.at[wid, j], srcv.at[0])
        pltpu.sync_copy(dsts.at[wid, j], dstv.at[0])
        pltpu.async_copy(table.at[srcv.at[0]], rows, sem).wait()
        pltpu.sync_copy(rows, acc.at[dstv.at[0]], add=True)
        pltpu.sync_copy(onesv, dacc.at[dstv.at[0]], add=True)
        return carry

    lax.fori_loop(0, _CHUNKS, step, 0)
    plsc.subcore_barrier()
    pltpu.sync_copy(acc.at[pl.ds(s * _XR, _XR)], part.at[c, pl.ds(s * _XR, _XR)])
    pltpu.sync_copy(dacc.at[pl.ds(s * _XR, _XR)], degp.at[c, pl.ds(s * _XR, _XR)])


_agg1 = pl.kernel(
    _agg1_body,
    out_type=[
        jax.ShapeDtypeStruct((_NC, _N, _D), jnp.float32),
        jax.ShapeDtypeStruct((_NC, _N, _DG), jnp.float32),
    ],
    mesh=_mesh,
    scratch_types=[
        pltpu.VMEM_SHARED((_ACC_ROWS, _D), jnp.float32),
        pltpu.VMEM_SHARED((_ACC_ROWS, _DG), jnp.float32),
        pltpu.VMEM((1, _K), jnp.int32),
        pltpu.VMEM((1, _K), jnp.int32),
        pltpu.VMEM((_K, _D), jnp.float32),
        pltpu.VMEM((_K, _DG), jnp.float32),
        pltpu.SemaphoreType.DMA,
    ],
)


def _agg2_body(table, srcs, dsts, zrows, part, acc, srcv, dstv, rows, sem):
    c = lax.axis_index("c")
    s = lax.axis_index("s")
    wid = c * _NS + s
    pltpu.sync_copy(zrows, acc.at[pl.ds(s * _ZR, _ZR)])
    plsc.subcore_barrier()

    def step(j, carry):
        pltpu.sync_copy(srcs.at[wid, j], srcv.at[0])
        pltpu.sync_copy(dsts.at[wid, j], dstv.at[0])
        pltpu.async_copy(table.at[srcv.at[0]], rows, sem).wait()
        pltpu.sync_copy(rows, acc.at[dstv.at[0]], add=True)
        return carry

    lax.fori_loop(0, _CHUNKS, step, 0)
    plsc.subcore_barrier()
    pltpu.sync_copy(acc.at[pl.ds(s * _XR, _XR)], part.at[c, pl.ds(s * _XR, _XR)])


_agg2 = pl.kernel(
    _agg2_body,
    out_type=jax.ShapeDtypeStruct((_NC, _N, _D), jnp.float32),
    mesh=_mesh,
    scratch_types=[
        pltpu.VMEM_SHARED((_ACC_ROWS, _D), jnp.float32),
        pltpu.VMEM((1, _K), jnp.int32),
        pltpu.VMEM((1, _K), jnp.int32),
        pltpu.VMEM((_K, _D), jnp.float32),
        pltpu.SemaphoreType.DMA,
    ],
)


def _dense_body(pref, dref, wref, bref, out_ref, *, relu):
    p = pref[0] + pref[1]
    d = dref[0, :, 0:1] + dref[1, :, 0:1]
    d = jnp.maximum(d, 1.0)
    h = jnp.dot(p * (1.0 / d), wref[...], preferred_element_type=jnp.float32)
    h = h + bref[...]
    if relu:
        h = jnp.maximum(h, 0.0)
    out_ref[...] = h


def _dense(part, degp, W, b2d, relu):
    bn = 1000
    return pl.pallas_call(
        functools.partial(_dense_body, relu=relu),
        grid=(_N // bn,),
        in_specs=[
            pl.BlockSpec((_NC, bn, _D), lambda i: (0, i, 0)),
            pl.BlockSpec((_NC, bn, _DG), lambda i: (0, i, 0)),
            pl.BlockSpec((_D, _D), lambda i: (0, 0)),
            pl.BlockSpec((1, _D), lambda i: (0, 0)),
        ],
        out_specs=pl.BlockSpec((bn, _D), lambda i: (i, 0)),
        out_shape=jax.ShapeDtypeStruct((_N, _D), jnp.float32),
    )(part, degp, W, b2d)


def kernel(x, adj_t, W1, b1, W2, b2):
    src = adj_t[0]
    dst = adj_t[1]
    pad = _EPAD - _E
    srcs = jnp.concatenate([src, jnp.zeros((pad,), jnp.int32)]).reshape(
        _NW, _CHUNKS, _K)
    dsts = jnp.concatenate([dst, jnp.full((pad,), _N, jnp.int32)]).reshape(
        _NW, _CHUNKS, _K)
    ones_h = jnp.ones((_K, _DG), jnp.float32)
    zrows = jnp.zeros((_ZR, _D), jnp.float32)
    zdeg = jnp.zeros((_ZR, _DG), jnp.float32)

    part1, degp = _agg1(x, srcs, dsts, ones_h, zrows, zdeg)
    h = _dense(part1, degp, W1, b1.reshape(1, _D), relu=True)
    part2 = _agg2(h, srcs, dsts, zrows)
    out = _dense(part2, degp, W2, b2.reshape(1, _D), relu=False)
    return out


# speedup vs baseline: 3.0791x; 3.0791x over previous
import functools

import jax
import jax.numpy as jnp
from jax import lax
from jax.experimental import pallas as pl
from jax.experimental.pallas import tpu as pltpu
from jax.experimental.pallas import tpu_sc as plsc

_N = 10000
_E = 320000
_D = 128
_NC = 2
_NS = 16
_NW = _NC * _NS
_K = 128
_CHUNKS = 80
_EPAD = _NW * _CHUNKS * _K
_ACC = 10240
_ZR = _ACC // _NS

_mesh = plsc.VectorSubcoreMesh(core_axis_name="c", subcore_axis_name="s")


def _agg1_body(table, srcs, dsts, zrows,
               part, deg, acc, dacc, srcv, dstv, rows, onesv, ztmp, sem):
    c = lax.axis_index("c")
    s = lax.axis_index("s")
    wid = c * _NS + s
    pltpu.sync_copy(zrows, acc.at[pl.ds(s * _ZR, _ZR)])

    def zz(r, carry):
        ztmp[pl.ds(r * 16, 16)] = jnp.zeros((16,), jnp.float32)
        return carry
    lax.fori_loop(0, _ZR // 16, zz, 0)

    def oo(r, carry):
        onesv[pl.ds(r * 16, 16)] = jnp.ones((16,), jnp.float32)
        return carry
    lax.fori_loop(0, _K // 16, oo, 0)

    pltpu.sync_copy(ztmp, dacc.at[pl.ds(s * _ZR, _ZR)])
    plsc.subcore_barrier()

    def step(j, carry):
        pltpu.sync_copy(srcs.at[wid, j], srcv.at[0])
        pltpu.sync_copy(dsts.at[wid, j], dstv.at[0])
        pltpu.async_copy(table.at[srcv.at[0]], rows, sem).wait()
        pltpu.sync_copy(rows, acc.at[dstv.at[0]], add=True)
        pltpu.sync_copy(onesv, dacc.at[dstv.at[0]], add=True)
        return carry

    lax.fori_loop(0, _CHUNKS, step, 0)
    plsc.subcore_barrier()
    pltpu.sync_copy(acc.at[pl.ds(s * _ZR, _ZR)], part.at[c, pl.ds(s * _ZR, _ZR)])
    pltpu.sync_copy(dacc.at[pl.ds(s * _ZR, _ZR)],
                    deg.at[pl.ds(c * _ACC + s * _ZR, _ZR)])


_agg1 = pl.kernel(
    _agg1_body,
    out_type=[
        jax.ShapeDtypeStruct((_NC, _ACC, _D), jnp.float32),
        jax.ShapeDtypeStruct((_NC * _ACC,), jnp.float32),
    ],
    mesh=_mesh,
    scratch_types=[
        pltpu.VMEM_SHARED((_ACC, _D), jnp.float32),
        pltpu.VMEM_SHARED((_ACC,), jnp.float32),
        pltpu.VMEM((1, _K), jnp.int32),
        pltpu.VMEM((1, _K), jnp.int32),
        pltpu.VMEM((_K, _D), jnp.float32),
        pltpu.VMEM((_K,), jnp.float32),
        pltpu.VMEM((_ZR,), jnp.float32),
        pltpu.SemaphoreType.DMA,
    ],
)


def _agg2_body(table, srcs, dsts, zrows, part, acc, srcv, dstv, rows, sem):
    c = lax.axis_index("c")
    s = lax.axis_index("s")
    wid = c * _NS + s
    pltpu.sync_copy(zrows, acc.at[pl.ds(s * _ZR, _ZR)])
    plsc.subcore_barrier()

    def step(j, carry):
        pltpu.sync_copy(srcs.at[wid, j], srcv.at[0])
        pltpu.sync_copy(dsts.at[wid, j], dstv.at[0])
        pltpu.async_copy(table.at[srcv.at[0]], rows, sem).wait()
        pltpu.sync_copy(rows, acc.at[dstv.at[0]], add=True)
        return carry

    lax.fori_loop(0, _CHUNKS, step, 0)
    plsc.subcore_barrier()
    pltpu.sync_copy(acc.at[pl.ds(s * _ZR, _ZR)], part.at[c, pl.ds(s * _ZR, _ZR)])


_agg2 = pl.kernel(
    _agg2_body,
    out_type=jax.ShapeDtypeStruct((_NC, _ACC, _D), jnp.float32),
    mesh=_mesh,
    scratch_types=[
        pltpu.VMEM_SHARED((_ACC, _D), jnp.float32),
        pltpu.VMEM((1, _K), jnp.int32),
        pltpu.VMEM((1, _K), jnp.int32),
        pltpu.VMEM((_K, _D), jnp.float32),
        pltpu.SemaphoreType.DMA,
    ],
)


def _dense_body(pref, dref, wref, bref, out_ref, *, relu):
    p = pref[0] + pref[1]
    d = dref[0] + dref[1]
    d = jnp.maximum(d, 1.0)
    h = jnp.dot(p * (1.0 / d), wref[...], preferred_element_type=jnp.float32)
    h = h + bref[...]
    if relu:
        h = jnp.maximum(h, 0.0)
    out_ref[...] = h


def _dense(part, deg3, W, b2d, relu):
    bn = 1000
    return pl.pallas_call(
        functools.partial(_dense_body, relu=relu),
        grid=(_N // bn,),
        in_specs=[
            pl.BlockSpec((_NC, bn, _D), lambda i: (0, i, 0)),
            pl.BlockSpec((_NC, bn, 1), lambda i: (0, i, 0)),
            pl.BlockSpec((_D, _D), lambda i: (0, 0)),
            pl.BlockSpec((1, _D), lambda i: (0, 0)),
        ],
        out_specs=pl.BlockSpec((bn, _D), lambda i: (i, 0)),
        out_shape=jax.ShapeDtypeStruct((_N, _D), jnp.float32),
    )(part, deg3, W, b2d)


def kernel(x, adj_t, W1, b1, W2, b2):
    src = adj_t[0]
    dst = adj_t[1]
    pad = _EPAD - _E
    srcs = jnp.concatenate([src, jnp.zeros((pad,), jnp.int32)]).reshape(
        _NW, _CHUNKS, _K)
    dsts = jnp.concatenate([dst, jnp.full((pad,), _N, jnp.int32)]).reshape(
        _NW, _CHUNKS, _K)
    zrows = jnp.zeros((_ZR, _D), jnp.float32)

    part1, deg = _agg1(x, srcs, dsts, zrows)
    deg3 = deg.reshape(_NC, _ACC, 1)
    h = _dense(part1, deg3, W1, b1.reshape(1, _D), relu=True)
    part2 = _agg2(h, srcs, dsts, zrows)
    out = _dense(part2, deg3, W2, b2.reshape(1, _D), relu=False)
    return out


# speedup vs baseline: 8.7956x; 2.8565x over previous
import functools

import jax
import jax.numpy as jnp
from jax import lax
from jax.experimental import pallas as pl
from jax.experimental.pallas import tpu as pltpu
from jax.experimental.pallas import tpu_sc as plsc

_N = 10000
_E = 320000
_D = 128
_NC = 2
_NS = 16
_NW = _NC * _NS
_K = 128
_CHUNKS = 80
_EPAD = _NW * _CHUNKS * _K
_ACC = 10240
_ZR = _ACC // _NS
_TRIP_LAST = (_E - (_NW - 1) * _CHUNKS * _K + _K - 1) // _K

_mesh = plsc.VectorSubcoreMesh(core_axis_name="c", subcore_axis_name="s")


def _agg1_body(table, srcs, dsts, zrows,
               part, deg, acc, dacc, srcl, dstl, rows, onesv, ztmp, sem):
    c = lax.axis_index("c")
    s = lax.axis_index("s")
    wid = c * _NS + s
    ntrip = jnp.where(wid == _NW - 1, _TRIP_LAST, _CHUNKS)
    pltpu.sync_copy(srcs.at[wid], srcl)
    pltpu.sync_copy(dsts.at[wid], dstl)
    pltpu.sync_copy(zrows, acc.at[pl.ds(s * _ZR, _ZR)])

    def zz(r, carry):
        ztmp[pl.ds(r * 16, 16)] = jnp.zeros((16,), jnp.float32)
        return carry
    lax.fori_loop(0, _ZR // 16, zz, 0)

    def oo(r, carry):
        onesv[pl.ds(r * 16, 16)] = jnp.ones((16,), jnp.float32)
        return carry
    lax.fori_loop(0, _K // 16, oo, 0)

    pltpu.sync_copy(ztmp, dacc.at[pl.ds(s * _ZR, _ZR)])
    plsc.subcore_barrier()

    def step(j, carry):
        pltpu.async_copy(table.at[srcl.at[j]], rows, sem).wait()
        pltpu.sync_copy(rows, acc.at[dstl.at[j]], add=True)
        pltpu.sync_copy(onesv, dacc.at[dstl.at[j]], add=True)
        return carry

    lax.fori_loop(0, ntrip, step, 0)
    plsc.subcore_barrier()
    pltpu.sync_copy(acc.at[pl.ds(s * _ZR, _ZR)], part.at[c, pl.ds(s * _ZR, _ZR)])
    pltpu.sync_copy(dacc.at[pl.ds(s * _ZR, _ZR)],
                    deg.at[pl.ds(c * _ACC + s * _ZR, _ZR)])


_agg1 = pl.kernel(
    _agg1_body,
    out_type=[
        jax.ShapeDtypeStruct((_NC, _ACC, _D), jnp.float32),
        jax.ShapeDtypeStruct((_NC * _ACC,), jnp.float32),
    ],
    mesh=_mesh,
    scratch_types=[
        pltpu.VMEM_SHARED((_ACC, _D), jnp.float32),
        pltpu.VMEM_SHARED((_ACC,), jnp.float32),
        pltpu.VMEM((_CHUNKS, _K), jnp.int32),
        pltpu.VMEM((_CHUNKS, _K), jnp.int32),
        pltpu.VMEM((_K, _D), jnp.float32),
        pltpu.VMEM((_K,), jnp.float32),
        pltpu.VMEM((_ZR,), jnp.float32),
        pltpu.SemaphoreType.DMA,
    ],
)


def _agg2_body(table, srcs, dsts, zrows, part, acc, srcl, dstl, rows, sem):
    c = lax.axis_index("c")
    s = lax.axis_index("s")
    wid = c * _NS + s
    ntrip = jnp.where(wid == _NW - 1, _TRIP_LAST, _CHUNKS)
    pltpu.sync_copy(srcs.at[wid], srcl)
    pltpu.sync_copy(dsts.at[wid], dstl)
    pltpu.sync_copy(zrows, acc.at[pl.ds(s * _ZR, _ZR)])
    plsc.subcore_barrier()

    def step(j, carry):
        pltpu.async_copy(table.at[srcl.at[j]], rows, sem).wait()
        pltpu.sync_copy(rows, acc.at[dstl.at[j]], add=True)
        return carry

    lax.fori_loop(0, ntrip, step, 0)
    plsc.subcore_barrier()
    pltpu.sync_copy(acc.at[pl.ds(s * _ZR, _ZR)], part.at[c, pl.ds(s * _ZR, _ZR)])


_agg2 = pl.kernel(
    _agg2_body,
    out_type=jax.ShapeDtypeStruct((_NC, _ACC, _D), jnp.float32),
    mesh=_mesh,
    scratch_types=[
        pltpu.VMEM_SHARED((_ACC, _D), jnp.float32),
        pltpu.VMEM((_CHUNKS, _K), jnp.int32),
        pltpu.VMEM((_CHUNKS, _K), jnp.int32),
        pltpu.VMEM((_K, _D), jnp.float32),
        pltpu.SemaphoreType.DMA,
    ],
)


def _dense_body(pref, dref, wref, bref, out_ref, *, relu):
    p = pref[0] + pref[1]
    d = dref[0] + dref[1]
    d = jnp.maximum(d, 1.0)
    h = jnp.dot(p * (1.0 / d), wref[...], preferred_element_type=jnp.float32)
    h = h + bref[...]
    if relu:
        h = jnp.maximum(h, 0.0)
    out_ref[...] = h


def _dense(part, deg3, W, b2d, relu):
    bn = 1000
    return pl.pallas_call(
        functools.partial(_dense_body, relu=relu),
        grid=(_N // bn,),
        in_specs=[
            pl.BlockSpec((_NC, bn, _D), lambda i: (0, i, 0)),
            pl.BlockSpec((_NC, bn, 1), lambda i: (0, i, 0)),
            pl.BlockSpec((_D, _D), lambda i: (0, 0)),
            pl.BlockSpec((1, _D), lambda i: (0, 0)),
        ],
        out_specs=pl.BlockSpec((bn, _D), lambda i: (i, 0)),
        out_shape=jax.ShapeDtypeStruct((_N, _D), jnp.float32),
    )(part, deg3, W, b2d)


def kernel(x, adj_t, W1, b1, W2, b2):
    src = adj_t[0]
    dst = adj_t[1]
    pad = _EPAD - _E
    dead = _N + (jnp.arange(pad, dtype=jnp.int32) % (_ACC - _N))
    srcs = jnp.concatenate([src, jnp.zeros((pad,), jnp.int32)]).reshape(
        _NW, _CHUNKS, _K)
    dsts = jnp.concatenate([dst, dead]).reshape(_NW, _CHUNKS, _K)
    zrows = jnp.zeros((_ZR, _D), jnp.float32)

    part1, deg = _agg1(x, srcs, dsts, zrows)
    deg3 = deg.reshape(_NC, _ACC, 1)
    h = _dense(part1, deg3, W1, b1.reshape(1, _D), relu=True)
    part2 = _agg2(h, srcs, dsts, zrows)
    out = _dense(part2, deg3, W2, b2.reshape(1, _D), relu=False)
    return out


# speedup vs baseline: 10.1517x; 1.1542x over previous
import functools

import jax
import jax.numpy as jnp
from jax import lax
from jax.experimental import pallas as pl
from jax.experimental.pallas import tpu as pltpu
from jax.experimental.pallas import tpu_sc as plsc

_N = 10000
_E = 320000
_D = 128
_NC = 2
_NS = 16
_NW = _NC * _NS
_K = 128
_CHUNKS = 80
_EPAD = _NW * _CHUNKS * _K
_ACC = 10240
_ZR = _ACC // _NS
_TRIP_LAST = (_E - (_NW - 1) * _CHUNKS * _K + _K - 1) // _K
_IB = _CHUNKS // 2

_mesh = plsc.VectorSubcoreMesh(core_axis_name="c", subcore_axis_name="s")


def _agg1_body(table, srcs, dsts, zrows, part, deg, acc, dacc, srcl, dstl,
               rows0, rows1, onesv, ztmp, sg0, sg1, ss0, ss1, sd):
    c = lax.axis_index("c")
    s = lax.axis_index("s")
    wid = c * _NS + s
    ntrip = jnp.where(wid == _NW - 1, _TRIP_LAST, _CHUNKS)
    pltpu.sync_copy(zrows, acc.at[pl.ds(s * _ZR, _ZR)])

    def zz(r, carry):
        ztmp[pl.ds(r * 16, 16)] = jnp.zeros((16,), jnp.float32)
        return carry
    lax.fori_loop(0, _ZR // 16, zz, 0)

    def oo(r, carry):
        onesv[pl.ds(r * 16, 16)] = jnp.ones((16,), jnp.float32)
        return carry
    lax.fori_loop(0, _K // 16, oo, 0)

    pltpu.sync_copy(ztmp, dacc.at[pl.ds(s * _ZR, _ZR)])
    plsc.subcore_barrier()

    def block(base, nblk):
        pltpu.sync_copy(srcs.at[wid, pl.ds(base, _IB)], srcl)
        pltpu.sync_copy(dsts.at[wid, pl.ds(base, _IB)], dstl)

        def pair(t, carry):
            j0 = 2 * t
            j1 = j0 + 1
            g0 = pltpu.async_copy(table.at[srcl.at[j0]], rows0, sg0)
            g1 = pltpu.async_copy(table.at[srcl.at[j1]], rows1, sg1)
            g0.wait()
            s0 = pltpu.async_copy(rows0, acc.at[dstl.at[j0]], ss0, add=True)
            d0 = pltpu.async_copy(onesv, dacc.at[dstl.at[j0]], sd, add=True)
            g1.wait()
            s1 = pltpu.async_copy(rows1, acc.at[dstl.at[j1]], ss1, add=True)
            d1 = pltpu.async_copy(onesv, dacc.at[dstl.at[j1]], sd, add=True)
            s0.wait()
            s1.wait()
            d0.wait()
            d1.wait()
            return carry

        lax.fori_loop(0, nblk // 2, pair, 0)

    nb0 = jnp.minimum(ntrip, _IB)
    block(0, nb0)
    block(_IB, ntrip - nb0)
    plsc.subcore_barrier()
    pltpu.sync_copy(acc.at[pl.ds(s * _ZR, _ZR)], part.at[c, pl.ds(s * _ZR, _ZR)])
    pltpu.sync_copy(dacc.at[pl.ds(s * _ZR, _ZR)],
                    deg.at[pl.ds(c * _ACC + s * _ZR, _ZR)])


_agg1 = pl.kernel(
    _agg1_body,
    out_type=[
        jax.ShapeDtypeStruct((_NC, _ACC, _D), jnp.float32),
        jax.ShapeDtypeStruct((_NC * _ACC,), jnp.float32),
    ],
    mesh=_mesh,
    scratch_types=[
        pltpu.VMEM_SHARED((_ACC, _D), jnp.float32),
        pltpu.VMEM_SHARED((_ACC,), jnp.float32),
        pltpu.VMEM((_IB, _K), jnp.int32),
        pltpu.VMEM((_IB, _K), jnp.int32),
        pltpu.VMEM((_K, _D), jnp.float32),
        pltpu.VMEM((_K, _D), jnp.float32),
        pltpu.VMEM((_K,), jnp.float32),
        pltpu.VMEM((_ZR,), jnp.float32),
        pltpu.SemaphoreType.DMA,
        pltpu.SemaphoreType.DMA,
        pltpu.SemaphoreType.DMA,
        pltpu.SemaphoreType.DMA,
        pltpu.SemaphoreType.DMA,
    ],
)


def _agg2_body(table, srcs, dsts, zrows, part, acc, srcl, dstl,
               rows0, rows1, sg0, sg1, ss0, ss1):
    c = lax.axis_index("c")
    s = lax.axis_index("s")
    wid = c * _NS + s
    ntrip = jnp.where(wid == _NW - 1, _TRIP_LAST, _CHUNKS)
    pltpu.sync_copy(zrows, acc.at[pl.ds(s * _ZR, _ZR)])
    plsc.subcore_barrier()

    def block(base, nblk):
        pltpu.sync_copy(srcs.at[wid, pl.ds(base, _IB)], srcl)
        pltpu.sync_copy(dsts.at[wid, pl.ds(base, _IB)], dstl)

        def pair(t, carry):
            j0 = 2 * t
            j1 = j0 + 1
            g0 = pltpu.async_copy(table.at[srcl.at[j0]], rows0, sg0)
            g1 = pltpu.async_copy(table.at[srcl.at[j1]], rows1, sg1)
            g0.wait()
            s0 = pltpu.async_copy(rows0, acc.at[dstl.at[j0]], ss0, add=True)
            g1.wait()
            s1 = pltpu.async_copy(rows1, acc.at[dstl.at[j1]], ss1, add=True)
            s0.wait()
            s1.wait()
            return carry

        lax.fori_loop(0, nblk // 2, pair, 0)

    nb0 = jnp.minimum(ntrip, _IB)
    block(0, nb0)
    block(_IB, ntrip - nb0)
    plsc.subcore_barrier()
    pltpu.sync_copy(acc.at[pl.ds(s * _ZR, _ZR)], part.at[c, pl.ds(s * _ZR, _ZR)])


_agg2 = pl.kernel(
    _agg2_body,
    out_type=jax.ShapeDtypeStruct((_NC, _ACC, _D), jnp.float32),
    mesh=_mesh,
    scratch_types=[
        pltpu.VMEM_SHARED((_ACC, _D), jnp.float32),
        pltpu.VMEM((_IB, _K), jnp.int32),
        pltpu.VMEM((_IB, _K), jnp.int32),
        pltpu.VMEM((_K, _D), jnp.float32),
        pltpu.VMEM((_K, _D), jnp.float32),
        pltpu.SemaphoreType.DMA,
        pltpu.SemaphoreType.DMA,
        pltpu.SemaphoreType.DMA,
        pltpu.SemaphoreType.DMA,
    ],
)


def _dense_body(pref, dref, wref, bref, out_ref, *, relu):
    p = pref[0] + pref[1]
    d = dref[0] + dref[1]
    d = jnp.maximum(d, 1.0)
    h = jnp.dot(p * (1.0 / d), wref[...], preferred_element_type=jnp.float32)
    h = h + bref[...]
    if relu:
        h = jnp.maximum(h, 0.0)
    out_ref[...] = h


def _dense(part, deg3, W, b2d, relu):
    bn = 1000
    return pl.pallas_call(
        functools.partial(_dense_body, relu=relu),
        grid=(_N // bn,),
        in_specs=[
            pl.BlockSpec((_NC, bn, _D), lambda i: (0, i, 0)),
            pl.BlockSpec((_NC, bn, 1), lambda i: (0, i, 0)),
            pl.BlockSpec((_D, _D), lambda i: (0, 0)),
            pl.BlockSpec((1, _D), lambda i: (0, 0)),
        ],
        out_specs=pl.BlockSpec((bn, _D), lambda i: (i, 0)),
        out_shape=jax.ShapeDtypeStruct((_N, _D), jnp.float32),
    )(part, deg3, W, b2d)


def kernel(x, adj_t, W1, b1, W2, b2):
    src = adj_t[0]
    dst = adj_t[1]
    pad = _EPAD - _E
    dead = _N + (jnp.arange(pad, dtype=jnp.int32) % (_ACC - _N))
    srcs = jnp.concatenate([src, jnp.zeros((pad,), jnp.int32)]).reshape(
        _NW, _CHUNKS, _K)
    dsts = jnp.concatenate([dst, dead]).reshape(_NW, _CHUNKS, _K)
    zrows = jnp.zeros((_ZR, _D), jnp.float32)

    part1, deg = _agg1(x, srcs, dsts, zrows)
    deg3 = deg.reshape(_NC, _ACC, 1)
    h = _dense(part1, deg3, W1, b1.reshape(1, _D), relu=True)
    part2 = _agg2(h, srcs, dsts, zrows)
    out = _dense(part2, deg3, W2, b2.reshape(1, _D), relu=False)
    return out


# speedup vs baseline: 10.5866x; 1.0428x over previous
import functools

import jax
import jax.numpy as jnp
from jax import lax
from jax.experimental import pallas as pl
from jax.experimental.pallas import tpu as pltpu
from jax.experimental.pallas import tpu_sc as plsc

_N = 10000
_E = 320000
_D = 128
_NC = 2
_NS = 16
_NW = _NC * _NS
_K = 128
_CHUNKS = 80
_EPAD = _NW * _CHUNKS * _K
_ACC = 10240
_ZR = _ACC // _NS
_TRIP_LAST = (_E - (_NW - 1) * _CHUNKS * _K + _K - 1) // _K
_IB = _CHUNKS // 2

_mesh = plsc.VectorSubcoreMesh(core_axis_name="c", subcore_axis_name="s")


def _agg1_body(table, srcs, dsts, zrows, part, deg, acc, dacc, srcl, dstl,
               rows0, rows1, onesv, ztmp, sg0, sg1, ss0, ss1, sd):
    c = lax.axis_index("c")
    s = lax.axis_index("s")
    wid = c * _NS + s
    ntrip = jnp.where(wid == _NW - 1, _TRIP_LAST, _CHUNKS)
    pltpu.sync_copy(zrows, acc.at[pl.ds(s * _ZR, _ZR)])

    def zz(r, carry):
        ztmp[pl.ds(r * 16, 16)] = jnp.zeros((16,), jnp.float32)
        return carry
    lax.fori_loop(0, _ZR // 16, zz, 0)

    def oo(r, carry):
        onesv[pl.ds(r * 16, 16)] = jnp.ones((16,), jnp.float32)
        return carry
    lax.fori_loop(0, _K // 16, oo, 0)

    pltpu.sync_copy(ztmp, dacc.at[pl.ds(s * _ZR, _ZR)])
    plsc.subcore_barrier()

    def block(base, nblk):
        pltpu.sync_copy(srcs.at[wid, pl.ds(base, _IB)], srcl)
        pltpu.sync_copy(dsts.at[wid, pl.ds(base, _IB)], dstl)

        def quad(t, carry):
            j0 = 4 * t
            j1 = j0 + 1
            j2 = j0 + 2
            j3 = j0 + 3
            g0 = pltpu.async_copy(table.at[srcl.at[j0]], rows0, sg0)
            g1 = pltpu.async_copy(table.at[srcl.at[j1]], rows1, sg1)
            g0.wait()
            s0 = pltpu.async_copy(rows0, acc.at[dstl.at[j0]], ss0, add=True)
            d0 = pltpu.async_copy(onesv, dacc.at[dstl.at[j0]], sd, add=True)
            g1.wait()
            s1 = pltpu.async_copy(rows1, acc.at[dstl.at[j1]], ss1, add=True)
            d1 = pltpu.async_copy(onesv, dacc.at[dstl.at[j1]], sd, add=True)
            s0.wait()
            g2 = pltpu.async_copy(table.at[srcl.at[j2]], rows0, sg0)
            s1.wait()
            g3 = pltpu.async_copy(table.at[srcl.at[j3]], rows1, sg1)
            g2.wait()
            s2 = pltpu.async_copy(rows0, acc.at[dstl.at[j2]], ss0, add=True)
            d2 = pltpu.async_copy(onesv, dacc.at[dstl.at[j2]], sd, add=True)
            g3.wait()
            s3 = pltpu.async_copy(rows1, acc.at[dstl.at[j3]], ss1, add=True)
            d3 = pltpu.async_copy(onesv, dacc.at[dstl.at[j3]], sd, add=True)
            s2.wait()
            s3.wait()
            d0.wait()
            d1.wait()
            d2.wait()
            d3.wait()
            return carry

        lax.fori_loop(0, nblk // 4, quad, 0)

    nb0 = jnp.minimum(ntrip, _IB)
    block(0, nb0)
    block(_IB, ntrip - nb0)
    plsc.subcore_barrier()
    pltpu.sync_copy(acc.at[pl.ds(s * _ZR, _ZR)], part.at[c, pl.ds(s * _ZR, _ZR)])
    pltpu.sync_copy(dacc.at[pl.ds(s * _ZR, _ZR)],
                    deg.at[pl.ds(c * _ACC + s * _ZR, _ZR)])


_agg1 = pl.kernel(
    _agg1_body,
    out_type=[
        jax.ShapeDtypeStruct((_NC, _ACC, _D), jnp.float32),
        jax.ShapeDtypeStruct((_NC * _ACC,), jnp.float32),
    ],
    mesh=_mesh,
    scratch_types=[
        pltpu.VMEM_SHARED((_ACC, _D), jnp.float32),
        pltpu.VMEM_SHARED((_ACC,), jnp.float32),
        pltpu.VMEM((_IB, _K), jnp.int32),
        pltpu.VMEM((_IB, _K), jnp.int32),
        pltpu.VMEM((_K, _D), jnp.float32),
        pltpu.VMEM((_K, _D), jnp.float32),
        pltpu.VMEM((_K,), jnp.float32),
        pltpu.VMEM((_ZR,), jnp.float32),
        pltpu.SemaphoreType.DMA,
        pltpu.SemaphoreType.DMA,
        pltpu.SemaphoreType.DMA,
        pltpu.SemaphoreType.DMA,
        pltpu.SemaphoreType.DMA,
    ],
)


def _agg2_body(table, srcs, dsts, zrows, part, acc, srcl, dstl,
               rows0, rows1, sg0, sg1, ss0, ss1):
    c = lax.axis_index("c")
    s = lax.axis_index("s")
    wid = c * _NS + s
    ntrip = jnp.where(wid == _NW - 1, _TRIP_LAST, _CHUNKS)
    pltpu.sync_copy(zrows, acc.at[pl.ds(s * _ZR, _ZR)])
    plsc.subcore_barrier()

    def block(base, nblk):
        pltpu.sync_copy(srcs.at[wid, pl.ds(base, _IB)], srcl)
        pltpu.sync_copy(dsts.at[wid, pl.ds(base, _IB)], dstl)

        def quad(t, carry):
            j0 = 4 * t
            j1 = j0 + 1
            j2 = j0 + 2
            j3 = j0 + 3
            g0 = pltpu.async_copy(table.at[srcl.at[j0]], rows0, sg0)
            g1 = pltpu.async_copy(table.at[srcl.at[j1]], rows1, sg1)
            g0.wait()
            s0 = pltpu.async_copy(rows0, acc.at[dstl.at[j0]], ss0, add=True)
            g1.wait()
            s1 = pltpu.async_copy(rows1, acc.at[dstl.at[j1]], ss1, add=True)
            s0.wait()
            g2 = pltpu.async_copy(table.at[srcl.at[j2]], rows0, sg0)
            s1.wait()
            g3 = pltpu.async_copy(table.at[srcl.at[j3]], rows1, sg1)
            g2.wait()
            s2 = pltpu.async_copy(rows0, acc.at[dstl.at[j2]], ss0, add=True)
            g3.wait()
            s3 = pltpu.async_copy(rows1, acc.at[dstl.at[j3]], ss1, add=True)
            s2.wait()
            s3.wait()
            return carry

        lax.fori_loop(0, nblk // 4, quad, 0)

    nb0 = jnp.minimum(ntrip, _IB)
    block(0, nb0)
    block(_IB, ntrip - nb0)
    plsc.subcore_barrier()
    pltpu.sync_copy(acc.at[pl.ds(s * _ZR, _ZR)], part.at[c, pl.ds(s * _ZR, _ZR)])


_agg2 = pl.kernel(
    _agg2_body,
    out_type=jax.ShapeDtypeStruct((_NC, _ACC, _D), jnp.float32),
    mesh=_mesh,
    scratch_types=[
        pltpu.VMEM_SHARED((_ACC, _D), jnp.float32),
        pltpu.VMEM((_IB, _K), jnp.int32),
        pltpu.VMEM((_IB, _K), jnp.int32),
        pltpu.VMEM((_K, _D), jnp.float32),
        pltpu.VMEM((_K, _D), jnp.float32),
        pltpu.SemaphoreType.DMA,
        pltpu.SemaphoreType.DMA,
        pltpu.SemaphoreType.DMA,
        pltpu.SemaphoreType.DMA,
    ],
)


def _dense_body(pref, dref, wref, bref, out_ref, *, relu):
    p = pref[0] + pref[1]
    d = dref[0] + dref[1]
    d = jnp.maximum(d, 1.0)
    h = jnp.dot(p * (1.0 / d), wref[...], preferred_element_type=jnp.float32)
    h = h + bref[...]
    if relu:
        h = jnp.maximum(h, 0.0)
    out_ref[...] = h


def _dense(part, deg3, W, b2d, relu):
    bn = 1000
    return pl.pallas_call(
        functools.partial(_dense_body, relu=relu),
        grid=(_N // bn,),
        in_specs=[
            pl.BlockSpec((_NC, bn, _D), lambda i: (0, i, 0)),
            pl.BlockSpec((_NC, bn, 1), lambda i: (0, i, 0)),
            pl.BlockSpec((_D, _D), lambda i: (0, 0)),
            pl.BlockSpec((1, _D), lambda i: (0, 0)),
        ],
        out_specs=pl.BlockSpec((bn, _D), lambda i: (i, 0)),
        out_shape=jax.ShapeDtypeStruct((_N, _D), jnp.float32),
    )(part, deg3, W, b2d)


def kernel(x, adj_t, W1, b1, W2, b2):
    src = adj_t[0]
    dst = adj_t[1]
    pad = _EPAD - _E
    dead = _N + (jnp.arange(pad, dtype=jnp.int32) % (_ACC - _N))
    srcs = jnp.concatenate([src, jnp.zeros((pad,), jnp.int32)]).reshape(
        _NW, _CHUNKS, _K)
    dsts = jnp.concatenate([dst, dead]).reshape(_NW, _CHUNKS, _K)
    zrows = jnp.zeros((_ZR, _D), jnp.float32)

    part1, deg = _agg1(x, srcs, dsts, zrows)
    deg3 = deg.reshape(_NC, _ACC, 1)
    h = _dense(part1, deg3, W1, b1.reshape(1, _D), relu=True)
    part2 = _agg2(h, srcs, dsts, zrows)
    out = _dense(part2, deg3, W2, b2.reshape(1, _D), relu=False)
    return out


# speedup vs baseline: 12.0090x; 1.1344x over previous
import functools

import jax
import jax.numpy as jnp
from jax import lax
from jax.experimental import pallas as pl
from jax.experimental.pallas import tpu as pltpu
from jax.experimental.pallas import tpu_sc as plsc

_N = 10000
_E = 320000
_D = 128
_NC = 2
_NS = 16
_NW = _NC * _NS
_K = 128
_CHUNKS = 80
_EPAD = _NW * _CHUNKS * _K
_ACC = 10240
_ZR = _ACC // _NS
_TRIP_LAST = (_E - (_NW - 1) * _CHUNKS * _K + _K - 1) // _K
_IB = _CHUNKS // 2

_mesh = plsc.VectorSubcoreMesh(core_axis_name="c", subcore_axis_name="s")


def _ladder(count, table, acc, srcl, dstl, rows, sgs, sss, jbase,
            dacc=None, onesv=None, sd=None):
    g = [None] * count
    s = [None] * count
    d = []
    g[0] = pltpu.async_copy(table.at[srcl.at[jbase]], rows[0], sgs[0])
    g[1] = pltpu.async_copy(table.at[srcl.at[jbase + 1]], rows[1], sgs[1])
    for i in range(count):
        b = i % 2
        g[i].wait()
        s[i] = pltpu.async_copy(rows[b], acc.at[dstl.at[jbase + i]], sss[b],
                                add=True)
        if dacc is not None:
            d.append(pltpu.async_copy(onesv, dacc.at[dstl.at[jbase + i]], sd,
                                      add=True))
        if i + 2 < count:
            s[i].wait()
            g[i + 2] = pltpu.async_copy(table.at[srcl.at[jbase + i + 2]],
                                        rows[b], sgs[b])
    s[count - 2].wait()
    s[count - 1].wait()
    for di in d:
        di.wait()


def _agg1_body(table, srcs, dsts, zrows, part, deg, acc, dacc, srcl, dstl,
               rows0, rows1, onesv, ztmp, sg0, sg1, ss0, ss1, sd):
    c = lax.axis_index("c")
    s = lax.axis_index("s")
    wid = c * _NS + s
    ntrip = jnp.where(wid == _NW - 1, _TRIP_LAST, _CHUNKS)
    pltpu.sync_copy(zrows, acc.at[pl.ds(s * _ZR, _ZR)])

    def zz(r, carry):
        ztmp[pl.ds(r * 16, 16)] = jnp.zeros((16,), jnp.float32)
        return carry
    lax.fori_loop(0, _ZR // 16, zz, 0)

    def oo(r, carry):
        onesv[pl.ds(r * 16, 16)] = jnp.ones((16,), jnp.float32)
        return carry
    lax.fori_loop(0, _K // 16, oo, 0)

    pltpu.sync_copy(ztmp, dacc.at[pl.ds(s * _ZR, _ZR)])
    plsc.subcore_barrier()

    def block(base, nblk):
        pltpu.sync_copy(srcs.at[wid, pl.ds(base, _IB)], srcl)
        pltpu.sync_copy(dsts.at[wid, pl.ds(base, _IB)], dstl)

        def oct_(t, carry):
            _ladder(8, table, acc, srcl, dstl, (rows0, rows1), (sg0, sg1),
                    (ss0, ss1), 8 * t, dacc=dacc, onesv=onesv, sd=sd)
            return carry

        lax.fori_loop(0, nblk // 8, oct_, 0)

        @pl.when(nblk % 8 >= 4)
        def _():
            _ladder(4, table, acc, srcl, dstl, (rows0, rows1), (sg0, sg1),
                    (ss0, ss1), (nblk // 8) * 8, dacc=dacc, onesv=onesv,
                    sd=sd)

    nb0 = jnp.minimum(ntrip, _IB)
    block(0, nb0)
    block(_IB, ntrip - nb0)
    plsc.subcore_barrier()
    pltpu.sync_copy(acc.at[pl.ds(s * _ZR, _ZR)], part.at[c, pl.ds(s * _ZR, _ZR)])
    pltpu.sync_copy(dacc.at[pl.ds(s * _ZR, _ZR)],
                    deg.at[pl.ds(c * _ACC + s * _ZR, _ZR)])


_agg1 = pl.kernel(
    _agg1_body,
    out_type=[
        jax.ShapeDtypeStruct((_NC, _ACC, _D), jnp.float32),
        jax.ShapeDtypeStruct((_NC * _ACC,), jnp.float32),
    ],
    mesh=_mesh,
    scratch_types=[
        pltpu.VMEM_SHARED((_ACC, _D), jnp.float32),
        pltpu.VMEM_SHARED((_ACC,), jnp.float32),
        pltpu.VMEM((_IB, _K), jnp.int32),
        pltpu.VMEM((_IB, _K), jnp.int32),
        pltpu.VMEM((_K, _D), jnp.float32),
        pltpu.VMEM((_K, _D), jnp.float32),
        pltpu.VMEM((_K,), jnp.float32),
        pltpu.VMEM((_ZR,), jnp.float32),
        pltpu.SemaphoreType.DMA,
        pltpu.SemaphoreType.DMA,
        pltpu.SemaphoreType.DMA,
        pltpu.SemaphoreType.DMA,
        pltpu.SemaphoreType.DMA,
    ],
)


def _agg2_body(table, srcs, dsts, zrows, part, acc, srcl, dstl,
               rows0, rows1, sg0, sg1, ss0, ss1):
    c = lax.axis_index("c")
    s = lax.axis_index("s")
    wid = c * _NS + s
    ntrip = jnp.where(wid == _NW - 1, _TRIP_LAST, _CHUNKS)
    pltpu.sync_copy(zrows, acc.at[pl.ds(s * _ZR, _ZR)])
    plsc.subcore_barrier()

    def block(base, nblk):
        pltpu.sync_copy(srcs.at[wid, pl.ds(base, _IB)], srcl)
        pltpu.sync_copy(dsts.at[wid, pl.ds(base, _IB)], dstl)

        def oct_(t, carry):
            _ladder(8, table, acc, srcl, dstl, (rows0, rows1), (sg0, sg1),
                    (ss0, ss1), 8 * t)
            return carry

        lax.fori_loop(0, nblk // 8, oct_, 0)

        @pl.when(nblk % 8 >= 4)
        def _():
            _ladder(4, table, acc, srcl, dstl, (rows0, rows1), (sg0, sg1),
                    (ss0, ss1), (nblk // 8) * 8)

    nb0 = jnp.minimum(ntrip, _IB)
    block(0, nb0)
    block(_IB, ntrip - nb0)
    plsc.subcore_barrier()
    pltpu.sync_copy(acc.at[pl.ds(s * _ZR, _ZR)], part.at[c, pl.ds(s * _ZR, _ZR)])


_agg2 = pl.kernel(
    _agg2_body,
    out_type=jax.ShapeDtypeStruct((_NC, _ACC, _D), jnp.float32),
    mesh=_mesh,
    scratch_types=[
        pltpu.VMEM_SHARED((_ACC, _D), jnp.float32),
        pltpu.VMEM((_IB, _K), jnp.int32),
        pltpu.VMEM((_IB, _K), jnp.int32),
        pltpu.VMEM((_K, _D), jnp.float32),
        pltpu.VMEM((_K, _D), jnp.float32),
        pltpu.SemaphoreType.DMA,
        pltpu.SemaphoreType.DMA,
        pltpu.SemaphoreType.DMA,
        pltpu.SemaphoreType.DMA,
    ],
)


def _dense_body(pref, dref, wref, bref, out_ref, *, relu):
    p = pref[0] + pref[1]
    d = dref[0] + dref[1]
    d = jnp.maximum(d, 1.0)
    h = jnp.dot(p * (1.0 / d), wref[...], preferred_element_type=jnp.float32)
    h = h + bref[...]
    if relu:
        h = jnp.maximum(h, 0.0)
    out_ref[...] = h


def _dense(part, deg3, W, b2d, relu):
    bn = 1000
    return pl.pallas_call(
        functools.partial(_dense_body, relu=relu),
        grid=(_N // bn,),
        in_specs=[
            pl.BlockSpec((_NC, bn, _D), lambda i: (0, i, 0)),
            pl.BlockSpec((_NC, bn, 1), lambda i: (0, i, 0)),
            pl.BlockSpec((_D, _D), lambda i: (0, 0)),
            pl.BlockSpec((1, _D), lambda i: (0, 0)),
        ],
        out_specs=pl.BlockSpec((bn, _D), lambda i: (i, 0)),
        out_shape=jax.ShapeDtypeStruct((_N, _D), jnp.float32),
    )(part, deg3, W, b2d)


def kernel(x, adj_t, W1, b1, W2, b2):
    src = adj_t[0]
    dst = adj_t[1]
    pad = _EPAD - _E
    dead = _N + (jnp.arange(pad, dtype=jnp.int32) % (_ACC - _N))
    srcs = jnp.concatenate([src, jnp.zeros((pad,), jnp.int32)]).reshape(
        _NW, _CHUNKS, _K)
    dsts = jnp.concatenate([dst, dead]).reshape(_NW, _CHUNKS, _K)
    zrows = jnp.zeros((_ZR, _D), jnp.float32)

    part1, deg = _agg1(x, srcs, dsts, zrows)
    deg3 = deg.reshape(_NC, _ACC, 1)
    h = _dense(part1, deg3, W1, b1.reshape(1, _D), relu=True)
    part2 = _agg2(h, srcs, dsts, zrows)
    out = _dense(part2, deg3, W2, b2.reshape(1, _D), relu=False)
    return out
